# baseline (speedup 1.0000x reference)
import functools

import jax
import jax.numpy as jnp
from jax.experimental import pallas as pl
from jax.experimental.pallas import tpu as pltpu

_GAP = 1.0
_THR = 0.1
_D = 256
_GB = 32
_CORES = 2
_BIG = 2.0 ** 100


def _loss_kernel(lz_ref, lm_ref, zr_ref, mr_ref, out_ref, acc_ref, *,
                 nsteps):
    j = pl.program_id(1)

    lhs_z = lz_ref[0]
    lhs_m = lm_ref[0]
    zrows = zr_ref[0]
    mrows = mr_ref[:, 0]

    h1 = jax.lax.broadcasted_iota(jnp.int32, (_GB + 1, _D), 0)
    h3 = jax.lax.broadcasted_iota(jnp.int32, (3 * _GB + 3, _D), 0)

    @pl.when(j == 0)
    def _init():
        acc_ref[...] = jnp.zeros_like(acc_ref)

    acc = acc_ref[...]
    dn = (((0,), (0,)), ((), ()))

    for g in range(_GB):
        sel_z = jnp.where(h1 == g, 1.0, 0.0)
        rhs_z = jnp.where(h1 == _GB, zrows[g:g + 1, :], sel_z)
        z = jax.lax.dot_general(lhs_z, rhs_z.astype(jnp.bfloat16), dn,
                                preferred_element_type=jnp.float32)

        sel_m = jnp.where((h3 % _GB) == g, _BIG, 0.0)
        sel_m = jnp.where(h3 >= 3 * _GB, 0.0, sel_m)
        rhs_m = sel_m
        rhs_m = jnp.where(h3 == 3 * _GB, mrows[0, g:g + 1, :], rhs_m)
        rhs_m = jnp.where(h3 == 3 * _GB + 1, mrows[1, g:g + 1, :], rhs_m)
        rhs_m = jnp.where(h3 == 3 * _GB + 2, mrows[2, g:g + 1, :], rhs_m)
        mp = jax.lax.dot_general(lhs_m, rhs_m.astype(jnp.bfloat16), dn,
                                 preferred_element_type=jnp.float32)

        s = jnp.maximum(jnp.minimum(z, mp), 0.0)
        acc = acc + jnp.sum(s.reshape(_D // _GB, _GB, _D), axis=0)
    acc_ref[...] = acc

    @pl.when(j == nsteps - 1)
    def _fin():
        out_ref[...] = jnp.sum(acc_ref[...]).reshape(1, 1, 1)


def _hi16(x):
    xi = jax.lax.bitcast_convert_type(x, jnp.uint32)
    f = jax.lax.bitcast_convert_type(xi & jnp.uint32(0xFFFF0000), jnp.float32)
    b = jax.lax.bitcast_convert_type((xi >> 16).astype(jnp.uint16),
                                     jnp.bfloat16)
    return f, b


def _split3(x):
    hf, hb = _hi16(x)
    r = x - hf
    mf, mb = _hi16(r)
    l = r - mf
    lb = jax.lax.bitcast_convert_type(
        (jax.lax.bitcast_convert_type(l, jnp.uint32) >> 16).astype(jnp.uint16),
        jnp.bfloat16)
    return hb, mb, lb


def kernel(input, gdt_ts):
    B = input.shape[0]
    K = B // _D
    G = K - 1
    N = G * _D * (_D - 1)
    nblk = K // _GB
    nsteps = nblk // _CORES
    scale = 2.0 / float(N)

    o2 = input.reshape(K, _D)
    t2 = gdt_ts.reshape(K, _D)
    t2 = jnp.where(jax.lax.broadcasted_iota(jnp.int32, (K, 1), 0) < G,
                   t2, 0.0)

    ones = jnp.ones((nblk, 1, _D), jnp.bfloat16)
    ob = o2.astype(jnp.bfloat16).reshape(nblk, _GB, _D)
    lz = jnp.concatenate([ob, ones], axis=1)
    th, tm, tl = _split3(t2)
    lm = jnp.concatenate(
        [th.reshape(nblk, _GB, _D), tm.reshape(nblk, _GB, _D),
         tl.reshape(nblk, _GB, _D), ones, ones, ones], axis=1)
    zr = (_GAP - o2).reshape(nblk, _GB, _D)
    rh, rm, rl = _split3(-_THR - t2)
    mr = jnp.stack([
        rh.astype(jnp.float32) * _BIG,
        rm.astype(jnp.float32) * _BIG,
        rl.astype(jnp.float32) * _BIG,
    ]).reshape(3, nblk, _GB, _D)

    body = functools.partial(_loss_kernel, nsteps=nsteps)

    idx = lambda i, j: (i * (nblk // _CORES) + j, 0, 0)
    parts = pl.pallas_call(
        body,
        grid=(_CORES, nsteps),
        in_specs=[
            pl.BlockSpec((1, _GB + 1, _D), idx),
            pl.BlockSpec((1, 3 * _GB + 3, _D), idx),
            pl.BlockSpec((1, _GB, _D), idx),
            pl.BlockSpec((3, 1, _GB, _D),
                         lambda i, j: (0, i * (nblk // _CORES) + j, 0, 0)),
        ],
        out_specs=pl.BlockSpec((1, 1, 1), lambda i, j: (i, 0, 0)),
        out_shape=jax.ShapeDtypeStruct((_CORES, 1, 1), jnp.float32),
        scratch_shapes=[pltpu.VMEM((_GB, _D), jnp.float32)],
        compiler_params=pltpu.CompilerParams(
            dimension_semantics=("parallel", "arbitrary"),
        ),
    )(lz, lm, zr, mr)

    return (jnp.sum(parts) * scale).reshape(1)


# speedup vs baseline: 1.1159x; 1.1159x over previous
import functools

import jax
import jax.numpy as jnp
from jax.experimental import pallas as pl
from jax.experimental.pallas import tpu as pltpu

_GAP = 1.0
_THR = 0.1
_D = 256
_GB = 32
_CORES = 2
_BIG = 2.0 ** 100


def _split3(x):
    h = x.astype(jnp.bfloat16)
    r = x - h.astype(jnp.float32)
    m = r.astype(jnp.bfloat16)
    l = (r - m.astype(jnp.float32)).astype(jnp.bfloat16)
    return h, m, l


def _loss_kernel(o_ref, t_ref, out_ref, acc_ref, *, nsteps, g_valid, scale):
    j = pl.program_id(1)
    core = pl.program_id(0)
    blk = core * nsteps + j

    o = o_ref[...]
    t = t_ref[...]

    row = blk * _GB + jax.lax.broadcasted_iota(jnp.int32, (_GB, 1), 0)
    t = jnp.where(row < g_valid, t, 0.0)

    ones_row = jnp.ones((1, _D), dtype=jnp.bfloat16)

    lhs_z = jnp.concatenate([o.astype(jnp.bfloat16), ones_row], axis=0)
    th, tm, tl = _split3(t)
    lhs_m = jnp.concatenate(
        [th, tm, tl, ones_row, ones_row, ones_row], axis=0)

    h9 = jax.lax.broadcasted_iota(jnp.int32, (_GB + 1, _D), 0)
    h27 = jax.lax.broadcasted_iota(jnp.int32, (3 * _GB + 3, _D), 0)

    @pl.when(j == 0)
    def _init():
        acc_ref[...] = jnp.zeros_like(acc_ref)

    acc = acc_ref[...]
    dn = (((0,), (0,)), ((), ()))
    for g in range(_GB):
        sel_z = jnp.where(h9 == g, 1.0, 0.0)
        rhs_z = jnp.where(h9 == _GB, _GAP - o[g:g + 1, :], sel_z)
        z = jax.lax.dot_general(lhs_z, rhs_z.astype(jnp.bfloat16), dn,
                                preferred_element_type=jnp.float32)

        rh, rm, rl = _split3(-_THR - t[g:g + 1, :])
        sel_m = jnp.where((h27 % _GB) == g, _BIG, 0.0)
        sel_m = jnp.where(h27 >= 3 * _GB, 0.0, sel_m)
        rhs_m = sel_m
        rhs_m = jnp.where(h27 == 3 * _GB, rh.astype(jnp.float32) * _BIG, rhs_m)
        rhs_m = jnp.where(h27 == 3 * _GB + 1, rm.astype(jnp.float32) * _BIG, rhs_m)
        rhs_m = jnp.where(h27 == 3 * _GB + 2, rl.astype(jnp.float32) * _BIG, rhs_m)
        mp = jax.lax.dot_general(lhs_m, rhs_m.astype(jnp.bfloat16), dn,
                                 preferred_element_type=jnp.float32)

        s = jnp.maximum(jnp.minimum(z, mp), 0.0)
        acc = acc + jnp.sum(s.reshape(_D // _GB, _GB, _D), axis=0)
    acc_ref[...] = acc

    @pl.when(j == nsteps - 1)
    def _fin():
        out_ref[...] = (jnp.sum(acc_ref[...]) * scale).reshape(1, 1, 1)


def kernel(input, gdt_ts):
    B = input.shape[0]
    K = B // _D
    G = K - 1
    N = G * _D * (_D - 1)

    o2 = input.reshape(K, _D)
    t2 = gdt_ts.reshape(K, _D)

    nsteps = K // (_CORES * _GB)
    scale = 2.0 / float(N)

    body = functools.partial(_loss_kernel, nsteps=nsteps, g_valid=G,
                             scale=scale)

    parts = pl.pallas_call(
        body,
        grid=(_CORES, nsteps),
        in_specs=[
            pl.BlockSpec((_GB, _D), lambda i, j: (i * (K // (_CORES * _GB)) + j, 0)),
            pl.BlockSpec((_GB, _D), lambda i, j: (i * (K // (_CORES * _GB)) + j, 0)),
        ],
        out_specs=pl.BlockSpec((1, 1, 1), lambda i, j: (i, 0, 0)),
        out_shape=jax.ShapeDtypeStruct((_CORES, 1, 1), jnp.float32),
        scratch_shapes=[pltpu.VMEM((_GB, _D), jnp.float32)],
        compiler_params=pltpu.CompilerParams(
            dimension_semantics=("parallel", "arbitrary"),
        ),
    )(o2, t2)

    return jnp.sum(parts).reshape(1)


# speedup vs baseline: 1.1526x; 1.0328x over previous
import functools

import jax
import jax.numpy as jnp
from jax.experimental import pallas as pl
from jax.experimental.pallas import tpu as pltpu

_GAP = 1.0
_THR = 0.1
_D = 256
_GB = 64
_CORES = 2
_BIG = 2.0 ** 100


def _split3(x):
    h = x.astype(jnp.bfloat16)
    r = x - h.astype(jnp.float32)
    m = r.astype(jnp.bfloat16)
    l = (r - m.astype(jnp.float32)).astype(jnp.bfloat16)
    return h, m, l


def _loss_kernel(o_ref, t_ref, out_ref, acc_ref, *, nsteps, g_valid, scale):
    j = pl.program_id(1)
    core = pl.program_id(0)
    blk = core * nsteps + j

    o = o_ref[...]
    t = t_ref[...]

    row = blk * _GB + jax.lax.broadcasted_iota(jnp.int32, (_GB, 1), 0)
    t = jnp.where(row < g_valid, t, 0.0)

    ones_row = jnp.ones((1, _D), dtype=jnp.bfloat16)

    lhs_z = jnp.concatenate([o.astype(jnp.bfloat16), ones_row], axis=0)
    th, tm, tl = _split3(t)
    lhs_m = jnp.concatenate(
        [th, tm, tl, ones_row, ones_row, ones_row], axis=0)

    h9 = jax.lax.broadcasted_iota(jnp.int32, (_GB + 1, _D), 0)
    h27 = jax.lax.broadcasted_iota(jnp.int32, (3 * _GB + 3, _D), 0)

    @pl.when(j == 0)
    def _init():
        acc_ref[...] = jnp.zeros_like(acc_ref)

    acc = acc_ref[...]
    dn = (((0,), (0,)), ((), ()))
    for g in range(_GB):
        sel_z = jnp.where(h9 == g, 1.0, 0.0)
        rhs_z = jnp.where(h9 == _GB, _GAP - o[g:g + 1, :], sel_z)
        z = jax.lax.dot_general(lhs_z, rhs_z.astype(jnp.bfloat16), dn,
                                preferred_element_type=jnp.float32)

        rh, rm, rl = _split3(-_THR - t[g:g + 1, :])
        sel_m = jnp.where((h27 % _GB) == g, _BIG, 0.0)
        sel_m = jnp.where(h27 >= 3 * _GB, 0.0, sel_m)
        rhs_m = sel_m
        rhs_m = jnp.where(h27 == 3 * _GB, rh.astype(jnp.float32) * _BIG, rhs_m)
        rhs_m = jnp.where(h27 == 3 * _GB + 1, rm.astype(jnp.float32) * _BIG, rhs_m)
        rhs_m = jnp.where(h27 == 3 * _GB + 2, rl.astype(jnp.float32) * _BIG, rhs_m)
        mp = jax.lax.dot_general(lhs_m, rhs_m.astype(jnp.bfloat16), dn,
                                 preferred_element_type=jnp.float32)

        s = jnp.maximum(jnp.minimum(z, mp), 0.0)
        acc = acc + jnp.sum(s.reshape(_D // _GB, _GB, _D), axis=0)
    acc_ref[...] = acc

    @pl.when(j == nsteps - 1)
    def _fin():
        out_ref[...] = (jnp.sum(acc_ref[...]) * scale).reshape(1, 1, 1)


def kernel(input, gdt_ts):
    B = input.shape[0]
    K = B // _D
    G = K - 1
    N = G * _D * (_D - 1)

    o2 = input.reshape(K, _D)
    t2 = gdt_ts.reshape(K, _D)

    nsteps = K // (_CORES * _GB)
    scale = 2.0 / float(N)

    body = functools.partial(_loss_kernel, nsteps=nsteps, g_valid=G,
                             scale=scale)

    parts = pl.pallas_call(
        body,
        grid=(_CORES, nsteps),
        in_specs=[
            pl.BlockSpec((_GB, _D), lambda i, j: (i * (K // (_CORES * _GB)) + j, 0)),
            pl.BlockSpec((_GB, _D), lambda i, j: (i * (K // (_CORES * _GB)) + j, 0)),
        ],
        out_specs=pl.BlockSpec((1, 1, 1), lambda i, j: (i, 0, 0)),
        out_shape=jax.ShapeDtypeStruct((_CORES, 1, 1), jnp.float32),
        scratch_shapes=[pltpu.VMEM((_GB, _D), jnp.float32)],
        compiler_params=pltpu.CompilerParams(
            dimension_semantics=("parallel", "arbitrary"),
        ),
    )(o2, t2)

    return jnp.sum(parts).reshape(1)


# speedup vs baseline: 1.1565x; 1.0034x over previous
import functools

import jax
import jax.numpy as jnp
from jax.experimental import pallas as pl
from jax.experimental.pallas import tpu as pltpu

_GAP = 1.0
_THR = 0.1
_D = 256
_GB = 64
_CORES = 2
_BIG = 2.0 ** 100


def _split3(x):
    h = x.astype(jnp.bfloat16)
    r = x - h.astype(jnp.float32)
    m = r.astype(jnp.bfloat16)
    l = (r - m.astype(jnp.float32)).astype(jnp.bfloat16)
    return h, m, l


def _loss_kernel(o_ref, t_ref, out_ref, acc_ref, *, nsteps, g_valid, scale):
    j = pl.program_id(1)
    core = pl.program_id(0)
    blk = core * nsteps + j

    o = o_ref[...]
    t = t_ref[...]

    row = blk * _GB + jax.lax.broadcasted_iota(jnp.int32, (_GB, 1), 0)
    t = jnp.where(row < g_valid, t, 0.0)

    ones_bf = jnp.ones((1, _D), dtype=jnp.bfloat16)
    big_row = jnp.full((1, _D), _BIG, dtype=jnp.bfloat16)

    ob = o.astype(jnp.bfloat16)
    zr = (_GAP - o).astype(jnp.bfloat16)
    th, tm, tl = _split3(t)
    rh, rm, rl = _split3(-_THR - t)
    m0 = (rh.astype(jnp.float32) * _BIG).astype(jnp.bfloat16)
    m1 = (rm.astype(jnp.float32) * _BIG).astype(jnp.bfloat16)
    m2 = (rl.astype(jnp.float32) * _BIG).astype(jnp.bfloat16)

    @pl.when(j == 0)
    def _init():
        acc_ref[...] = jnp.zeros_like(acc_ref)

    acc = acc_ref[...]
    dn = (((0,), (0,)), ((), ()))
    for g in range(_GB):
        lhs_zg = jnp.concatenate([ob[g:g + 1], ones_bf], axis=0)
        rhs_zg = jnp.concatenate([ones_bf, zr[g:g + 1]], axis=0)
        z = jax.lax.dot_general(lhs_zg, rhs_zg, dn,
                                preferred_element_type=jnp.float32)

        lhs_mg = jnp.concatenate(
            [th[g:g + 1], tm[g:g + 1], tl[g:g + 1],
             ones_bf, ones_bf, ones_bf], axis=0)
        rhs_mg = jnp.concatenate(
            [big_row, big_row, big_row,
             m0[g:g + 1], m1[g:g + 1], m2[g:g + 1]], axis=0)
        mp = jax.lax.dot_general(lhs_mg, rhs_mg, dn,
                                 preferred_element_type=jnp.float32)

        s = jnp.maximum(jnp.minimum(z, mp), 0.0)
        acc = acc + jnp.sum(s.reshape(_D // _GB, _GB, _D), axis=0)
    acc_ref[...] = acc

    @pl.when(j == nsteps - 1)
    def _fin():
        out_ref[...] = (jnp.sum(acc_ref[...]) * scale).reshape(1, 1, 1)


def kernel(input, gdt_ts):
    B = input.shape[0]
    K = B // _D
    G = K - 1
    N = G * _D * (_D - 1)

    o2 = input.reshape(K, _D)
    t2 = gdt_ts.reshape(K, _D)

    nsteps = K // (_CORES * _GB)
    scale = 2.0 / float(N)

    body = functools.partial(_loss_kernel, nsteps=nsteps, g_valid=G,
                             scale=scale)

    parts = pl.pallas_call(
        body,
        grid=(_CORES, nsteps),
        in_specs=[
            pl.BlockSpec((_GB, _D), lambda i, j: (i * (K // (_CORES * _GB)) + j, 0)),
            pl.BlockSpec((_GB, _D), lambda i, j: (i * (K // (_CORES * _GB)) + j, 0)),
        ],
        out_specs=pl.BlockSpec((1, 1, 1), lambda i, j: (i, 0, 0)),
        out_shape=jax.ShapeDtypeStruct((_CORES, 1, 1), jnp.float32),
        scratch_shapes=[pltpu.VMEM((_GB, _D), jnp.float32)],
        compiler_params=pltpu.CompilerParams(
            dimension_semantics=("parallel", "arbitrary"),
        ),
    )(o2, t2)

    return jnp.sum(parts).reshape(1)
